# Initial kernel scaffold; baseline (speedup 1.0000x reference)
import functools

import jax
import jax.numpy as jnp
from jax import lax
from jax.experimental import pallas as pl
from jax.experimental.pallas import tpu as pltpu
from jax.experimental.pallas import tpu_sc as plsc

_N = 10000
_E = 320000
_D = 128
_TAU0 = 0.1

_NPAD = 10240
_NW = 32
_ROWS = 80
_EPW = _ROWS * 128
_EP = _NW * _EPW


def _mesh():
    return plsc.VectorSubcoreMesh(core_axis_name="c", subcore_axis_name="s")


@functools.partial(
    pl.kernel,
    mesh=_mesh(),
    out_type=jax.ShapeDtypeStruct((2, _NPAD), jnp.float32),
    scratch_types=[
        pltpu.VMEM((_ROWS, 128), jnp.int32),
        pltpu.VMEM((_ROWS, 128), jnp.int32),
        pltpu.VMEM((_ROWS, 128), jnp.float32),
        pltpu.VMEM_SHARED((_NPAD,), jnp.float32),
    ],
)
def _sc_deg(src_hbm, dst_hbm, zeros_hbm, degp_hbm, src_v, dst_v, val_v, acc_sh):
    c = lax.axis_index("c")
    s = lax.axis_index("s")
    w = c * 16 + s

    @pl.when(s == 0)
    def _():
        pltpu.sync_copy(zeros_hbm, acc_sh)

    pltpu.sync_copy(src_hbm.at[w], src_v)
    pltpu.sync_copy(dst_hbm.at[w], dst_v)

    def row(i, carry):
        def lane(j, carry2):
            sv = src_v[i, pl.ds(j * 16, 16)]
            dv = dst_v[i, pl.ds(j * 16, 16)]
            val_v[i, pl.ds(j * 16, 16)] = jnp.where(sv != dv, 1.0, 0.0)
            return carry2

        return lax.fori_loop(0, 8, lane, carry)

    lax.fori_loop(0, _ROWS, row, 0)

    plsc.subcore_barrier()
    pltpu.sync_copy(val_v, acc_sh.at[dst_v], add=True)
    plsc.subcore_barrier()

    @pl.when(s == 0)
    def _():
        pltpu.sync_copy(acc_sh, degp_hbm.at[c])


@functools.partial(
    pl.kernel,
    mesh=_mesh(),
    out_type=jax.ShapeDtypeStruct((2, _NPAD), jnp.float32),
    scratch_types=[
        pltpu.VMEM((_ROWS, 128), jnp.int32),
        pltpu.VMEM((_ROWS, 128), jnp.int32),
        pltpu.VMEM((_ROWS, 128), jnp.float32),
        pltpu.VMEM((_NPAD,), jnp.float32),
        pltpu.VMEM((_NPAD,), jnp.float32),
        pltpu.VMEM_SHARED((_NPAD,), jnp.float32),
    ],
)
def _sc_agg(src_hbm, dst_hbm, z_hbm, dinv_hbm, zeros_hbm, outp_hbm,
            src_v, dst_v, val_v, z_v, dinv_v, acc_sh):
    c = lax.axis_index("c")
    s = lax.axis_index("s")
    w = c * 16 + s

    @pl.when(s == 0)
    def _():
        pltpu.sync_copy(zeros_hbm, acc_sh)

    pltpu.sync_copy(z_hbm, z_v)
    pltpu.sync_copy(dinv_hbm, dinv_v)
    pltpu.sync_copy(src_hbm.at[w], src_v)
    pltpu.sync_copy(dst_hbm.at[w], dst_v)

    def row(i, carry):
        def lane(j, carry2):
            sv = src_v[i, pl.ds(j * 16, 16)]
            dv = dst_v[i, pl.ds(j * 16, 16)]
            zz = plsc.load_gather(z_v, [sv])
            dd = plsc.load_gather(dinv_v, [dv])
            val_v[i, pl.ds(j * 16, 16)] = jnp.where(sv != dv, zz * dd, 0.0)
            return carry2

        return lax.fori_loop(0, 8, lane, carry)

    lax.fori_loop(0, _ROWS, row, 0)

    plsc.subcore_barrier()
    pltpu.sync_copy(val_v, acc_sh.at[dst_v], add=True)
    plsc.subcore_barrier()

    @pl.when(s == 0)
    def _():
        pltpu.sync_copy(acc_sh, outp_hbm.at[c])


def _tc_prep_body(degp_ref, xt_ref, w_ref, dinv_ref, z_ref):
    deg = degp_ref[0:1, :] + degp_ref[1:2, :]
    dinv = jnp.where(deg > 0.0, lax.rsqrt(deg), 0.0)
    y = jnp.sum(xt_ref[...] * w_ref[...], axis=0, keepdims=True)
    dinv_ref[...] = dinv
    z_ref[...] = dinv * y


_tc_prep = pl.pallas_call(
    _tc_prep_body,
    out_shape=(
        jax.ShapeDtypeStruct((1, _NPAD), jnp.float32),
        jax.ShapeDtypeStruct((1, _NPAD), jnp.float32),
    ),
)


def _tc_head_body(outp_ref, temp_ref):
    o = outp_ref[0:1, :] + outp_ref[1:2, :]
    sp = jnp.maximum(o, 0.0) + jnp.log1p(jnp.exp(-jnp.abs(o))) + _TAU0
    temp_ref[...] = 1.0 / sp


_tc_head = pl.pallas_call(
    _tc_head_body,
    out_shape=jax.ShapeDtypeStruct((1, _NPAD), jnp.float32),
)


def kernel(x, edge_index, W):
    src = edge_index[0]
    dst = edge_index[1]
    pad = _EP - _E
    srcp = jnp.pad(src, (0, pad)).reshape(_NW, _ROWS, 128)
    dstp = jnp.pad(dst, (0, pad)).reshape(_NW, _ROWS, 128)
    zeros = jnp.zeros((_NPAD,), jnp.float32)

    degp = _sc_deg(srcp, dstp, zeros)
    xt = jnp.pad(x, ((0, _NPAD - _N), (0, 0))).T
    dinv, z = _tc_prep(degp, xt, W)
    outp = _sc_agg(srcp, dstp, z.reshape(_NPAD), dinv.reshape(_NPAD), zeros)
    temp = _tc_head(outp)
    return temp[0, :_N].reshape(_N, 1)


# speedup vs baseline: 86.1130x; 86.1130x over previous
import functools

import jax
import jax.numpy as jnp
from jax import lax
from jax.experimental import pallas as pl
from jax.experimental.pallas import tpu as pltpu
from jax.experimental.pallas import tpu_sc as plsc

_N = 10000
_E = 320000
_D = 128
_TAU0 = 0.1

_NPAD = 10240
_NW = 32
_ROWS = 80
_EPW = _ROWS * 128
_EP = _NW * _EPW


def _mesh():
    return plsc.VectorSubcoreMesh(core_axis_name="c", subcore_axis_name="s")


@functools.partial(
    pl.kernel,
    mesh=_mesh(),
    out_type=jax.ShapeDtypeStruct((2, _NPAD), jnp.float32),
    scratch_types=[
        pltpu.VMEM((_ROWS, 128), jnp.int32),
        pltpu.VMEM((_ROWS, 128), jnp.int32),
        pltpu.VMEM((_ROWS, 128), jnp.float32),
        pltpu.VMEM_SHARED((_NPAD,), jnp.float32),
    ],
)
def _sc_deg(src_hbm, dst_hbm, zeros_hbm, degp_hbm, src_v, dst_v, val_v, acc_sh):
    c = lax.axis_index("c")
    s = lax.axis_index("s")
    w = c * 16 + s

    @pl.when(s == 0)
    def _():
        pltpu.sync_copy(zeros_hbm, acc_sh)

    pltpu.sync_copy(src_hbm.at[w], src_v)
    pltpu.sync_copy(dst_hbm.at[w], dst_v)

    def row(i, carry):
        def lane(j, carry2):
            sv = src_v[i, pl.ds(j * 16, 16)]
            dv = dst_v[i, pl.ds(j * 16, 16)]
            val_v[i, pl.ds(j * 16, 16)] = jnp.where(sv != dv, 1.0, 0.0)
            return carry2

        return lax.fori_loop(0, 8, lane, carry)

    lax.fori_loop(0, _ROWS, row, 0)

    plsc.subcore_barrier()

    def scat(i, carry):
        pltpu.sync_copy(val_v.at[i], acc_sh.at[dst_v.at[i]], add=True)
        return carry

    lax.fori_loop(0, _ROWS, scat, 0)
    plsc.subcore_barrier()

    @pl.when(s == 0)
    def _():
        pltpu.sync_copy(acc_sh, degp_hbm.at[c])


@functools.partial(
    pl.kernel,
    mesh=_mesh(),
    out_type=jax.ShapeDtypeStruct((2, _NPAD), jnp.float32),
    compiler_params=pltpu.CompilerParams(needs_layout_passes=False),
    scratch_types=[
        pltpu.VMEM((_ROWS, 128), jnp.int32),
        pltpu.VMEM((_ROWS, 128), jnp.int32),
        pltpu.VMEM((_ROWS, 128), jnp.float32),
        pltpu.VMEM((_NPAD,), jnp.float32),
        pltpu.VMEM((_NPAD,), jnp.float32),
        pltpu.VMEM_SHARED((_NPAD,), jnp.float32),
    ],
)
def _sc_agg(src_hbm, dst_hbm, z_hbm, dinv_hbm, zeros_hbm, outp_hbm,
            src_v, dst_v, val_v, z_v, dinv_v, acc_sh):
    c = lax.axis_index("c")
    s = lax.axis_index("s")
    w = c * 16 + s

    @pl.when(s == 0)
    def _():
        pltpu.sync_copy(zeros_hbm, acc_sh)

    pltpu.sync_copy(z_hbm, z_v)
    pltpu.sync_copy(dinv_hbm, dinv_v)
    pltpu.sync_copy(src_hbm.at[w], src_v)
    pltpu.sync_copy(dst_hbm.at[w], dst_v)

    def row(i, carry):
        src_r = src_v.at[i]
        dst_r = dst_v.at[i]
        val_r = val_v.at[i]

        def lane(j, carry2):
            sv = src_r[pl.ds(j * 16, 16)]
            dv = dst_r[pl.ds(j * 16, 16)]
            zz = plsc.load_gather(z_v, [sv])
            dd = plsc.load_gather(dinv_v, [dv])
            val_r[pl.ds(j * 16, 16)] = jnp.where(sv != dv, zz * dd, 0.0)
            return carry2

        return lax.fori_loop(0, 8, lane, carry)

    lax.fori_loop(0, _ROWS, row, 0)

    plsc.subcore_barrier()

    def scat(i, carry):
        pltpu.sync_copy(val_v.at[i], acc_sh.at[dst_v.at[i]], add=True)
        return carry

    lax.fori_loop(0, _ROWS, scat, 0)
    plsc.subcore_barrier()

    @pl.when(s == 0)
    def _():
        pltpu.sync_copy(acc_sh, outp_hbm.at[c])


def _tc_prep_body(degp_ref, xt_ref, w_ref, dinv_ref, z_ref):
    deg = degp_ref[0:1, :] + degp_ref[1:2, :]
    dinv = jnp.where(deg > 0.0, lax.rsqrt(deg), 0.0)
    y = jnp.sum(xt_ref[...] * w_ref[...], axis=0, keepdims=True)
    dinv_ref[...] = dinv
    z_ref[...] = dinv * y


_tc_prep = pl.pallas_call(
    _tc_prep_body,
    out_shape=(
        jax.ShapeDtypeStruct((1, _NPAD), jnp.float32),
        jax.ShapeDtypeStruct((1, _NPAD), jnp.float32),
    ),
)


def _tc_head_body(outp_ref, temp_ref):
    o = outp_ref[0:1, :] + outp_ref[1:2, :]
    sp = jnp.maximum(o, 0.0) + jnp.log1p(jnp.exp(-jnp.abs(o))) + _TAU0
    temp_ref[...] = 1.0 / sp


_tc_head = pl.pallas_call(
    _tc_head_body,
    out_shape=jax.ShapeDtypeStruct((1, _NPAD), jnp.float32),
)


def kernel(x, edge_index, W):
    src = edge_index[0]
    dst = edge_index[1]
    pad = _EP - _E
    srcp = jnp.pad(src, (0, pad)).reshape(_NW, _ROWS, 128)
    dstp = jnp.pad(dst, (0, pad)).reshape(_NW, _ROWS, 128)
    zeros = jnp.zeros((_NPAD,), jnp.float32)

    degp = _sc_deg(srcp, dstp, zeros)
    xt = jnp.pad(x, ((0, _NPAD - _N), (0, 0))).T
    dinv, z = _tc_prep(degp, xt, W)
    outp = _sc_agg(srcp, dstp, z.reshape(_NPAD), dinv.reshape(_NPAD), zeros)
    temp = _tc_head(outp)
    return temp[0, :_N].reshape(_N, 1)


# speedup vs baseline: 99.1666x; 1.1516x over previous
import functools

import jax
import jax.numpy as jnp
from jax import lax
from jax.experimental import pallas as pl
from jax.experimental.pallas import tpu as pltpu
from jax.experimental.pallas import tpu_sc as plsc

_N = 10000
_E = 320000
_D = 128
_TAU0 = 0.1

_NPAD = 10240
_NW = 32
_ROWS = 80
_EPW = _ROWS * 128
_EP = _NW * _EPW


def _mesh():
    return plsc.VectorSubcoreMesh(core_axis_name="c", subcore_axis_name="s")


@functools.partial(
    pl.kernel,
    mesh=_mesh(),
    out_type=jax.ShapeDtypeStruct((2, _NPAD), jnp.float32),
    scratch_types=[
        pltpu.VMEM((_ROWS, 128), jnp.int32),
        pltpu.VMEM((_ROWS, 128), jnp.int32),
        pltpu.VMEM((_ROWS, 128), jnp.float32),
        pltpu.VMEM_SHARED((_NPAD,), jnp.float32),
        pltpu.SemaphoreType.DMA,
        pltpu.SemaphoreType.DMA,
    ],
)
def _sc_deg(src_hbm, dst_hbm, zeros_hbm, degp_hbm,
            src_v, dst_v, val_v, acc_sh, dsem, ssem):
    c = lax.axis_index("c")
    s = lax.axis_index("s")
    w = c * 16 + s
    seg = pl.ds(s * (_NPAD // 16), _NPAD // 16)

    pltpu.async_copy(src_hbm.at[w], src_v, dsem)
    pltpu.async_copy(dst_hbm.at[w], dst_v, dsem)
    pltpu.sync_copy(zeros_hbm.at[seg], acc_sh.at[seg])
    plsc.subcore_barrier()
    pltpu.make_async_copy(src_hbm.at[w], src_v, dsem).wait()
    pltpu.make_async_copy(dst_hbm.at[w], dst_v, dsem).wait()

    def row(i, carry):
        src_r = src_v.at[i]
        dst_r = dst_v.at[i]
        val_r = val_v.at[i]

        def lane(j, carry2):
            sv = src_r[pl.ds(j * 16, 16)]
            dv = dst_r[pl.ds(j * 16, 16)]
            val_r[pl.ds(j * 16, 16)] = jnp.where(sv != dv, 1.0, 0.0)
            return carry2

        lax.fori_loop(0, 8, lane, carry)
        pltpu.async_copy(val_v.at[i], acc_sh.at[dst_v.at[i]], ssem, add=True)
        return carry

    lax.fori_loop(0, _ROWS, row, 0)

    def drain(i, carry):
        pltpu.make_async_copy(val_v.at[i], acc_sh.at[dst_v.at[i]], ssem).wait()
        return carry

    lax.fori_loop(0, _ROWS, drain, 0)
    plsc.subcore_barrier()
    pltpu.sync_copy(acc_sh.at[seg], degp_hbm.at[c].at[seg])


@functools.partial(
    pl.kernel,
    mesh=_mesh(),
    out_type=jax.ShapeDtypeStruct((2, _NPAD), jnp.float32),
    compiler_params=pltpu.CompilerParams(needs_layout_passes=False),
    scratch_types=[
        pltpu.VMEM((_ROWS, 128), jnp.int32),
        pltpu.VMEM((_ROWS, 128), jnp.int32),
        pltpu.VMEM((_ROWS, 128), jnp.float32),
        pltpu.VMEM((_NPAD,), jnp.float32),
        pltpu.VMEM((_NPAD,), jnp.float32),
        pltpu.VMEM_SHARED((_NPAD,), jnp.float32),
        pltpu.SemaphoreType.DMA,
        pltpu.SemaphoreType.DMA,
    ],
)
def _sc_agg(src_hbm, dst_hbm, z_hbm, dinv_hbm, zeros_hbm, outp_hbm,
            src_v, dst_v, val_v, z_v, dinv_v, acc_sh, dsem, ssem):
    c = lax.axis_index("c")
    s = lax.axis_index("s")
    w = c * 16 + s
    seg = pl.ds(s * (_NPAD // 16), _NPAD // 16)

    pltpu.async_copy(src_hbm.at[w], src_v, dsem)
    pltpu.async_copy(dst_hbm.at[w], dst_v, dsem)
    pltpu.async_copy(z_hbm, z_v, dsem)
    pltpu.async_copy(dinv_hbm, dinv_v, dsem)
    pltpu.sync_copy(zeros_hbm.at[seg], acc_sh.at[seg])
    plsc.subcore_barrier()
    pltpu.make_async_copy(src_hbm.at[w], src_v, dsem).wait()
    pltpu.make_async_copy(dst_hbm.at[w], dst_v, dsem).wait()
    pltpu.make_async_copy(z_hbm, z_v, dsem).wait()
    pltpu.make_async_copy(dinv_hbm, dinv_v, dsem).wait()

    def row(i, carry):
        src_r = src_v.at[i]
        dst_r = dst_v.at[i]
        val_r = val_v.at[i]

        def lane(j, carry2):
            sv = src_r[pl.ds(j * 16, 16)]
            dv = dst_r[pl.ds(j * 16, 16)]
            zz = plsc.load_gather(z_v, [sv])
            dd = plsc.load_gather(dinv_v, [dv])
            val_r[pl.ds(j * 16, 16)] = jnp.where(sv != dv, zz * dd, 0.0)
            return carry2

        lax.fori_loop(0, 8, lane, carry)
        pltpu.async_copy(val_v.at[i], acc_sh.at[dst_v.at[i]], ssem, add=True)
        return carry

    lax.fori_loop(0, _ROWS, row, 0)

    def drain(i, carry):
        pltpu.make_async_copy(val_v.at[i], acc_sh.at[dst_v.at[i]], ssem).wait()
        return carry

    lax.fori_loop(0, _ROWS, drain, 0)
    plsc.subcore_barrier()
    pltpu.sync_copy(acc_sh.at[seg], outp_hbm.at[c].at[seg])


def _tc_prep_body(degp_ref, xt_ref, w_ref, dinv_ref, z_ref):
    deg = degp_ref[0:1, :] + degp_ref[1:2, :]
    dinv = jnp.where(deg > 0.0, lax.rsqrt(deg), 0.0)
    y = jnp.sum(xt_ref[...] * w_ref[...], axis=0, keepdims=True)
    dinv_ref[...] = dinv
    z_ref[...] = dinv * y


_tc_prep = pl.pallas_call(
    _tc_prep_body,
    out_shape=(
        jax.ShapeDtypeStruct((1, _NPAD), jnp.float32),
        jax.ShapeDtypeStruct((1, _NPAD), jnp.float32),
    ),
)


def _tc_head_body(outp_ref, temp_ref):
    o = outp_ref[0:1, :] + outp_ref[1:2, :]
    sp = jnp.maximum(o, 0.0) + jnp.log1p(jnp.exp(-jnp.abs(o))) + _TAU0
    temp_ref[...] = 1.0 / sp


_tc_head = pl.pallas_call(
    _tc_head_body,
    out_shape=jax.ShapeDtypeStruct((1, _NPAD), jnp.float32),
)


def kernel(x, edge_index, W):
    src = edge_index[0]
    dst = edge_index[1]
    pad = _EP - _E
    srcp = jnp.pad(src, (0, pad)).reshape(_NW, _ROWS, 128)
    dstp = jnp.pad(dst, (0, pad)).reshape(_NW, _ROWS, 128)
    zeros = jnp.zeros((_NPAD,), jnp.float32)

    degp = _sc_deg(srcp, dstp, zeros)
    xt = jnp.pad(x, ((0, _NPAD - _N), (0, 0))).T
    dinv, z = _tc_prep(degp, xt, W)
    outp = _sc_agg(srcp, dstp, z.reshape(_NPAD), dinv.reshape(_NPAD), zeros)
    temp = _tc_head(outp)
    return temp[0, :_N].reshape(_N, 1)


# speedup vs baseline: 100.9185x; 1.0177x over previous
import functools

import jax
import jax.numpy as jnp
from jax import lax
from jax.experimental import pallas as pl
from jax.experimental.pallas import tpu as pltpu
from jax.experimental.pallas import tpu_sc as plsc

_N = 10000
_E = 320000
_D = 128
_TAU0 = 0.1

_NPAD = 10240
_NW = 32
_ROWS = 80
_EPW = _ROWS * 128
_EP = _NW * _EPW


def _mesh():
    return plsc.VectorSubcoreMesh(core_axis_name="c", subcore_axis_name="s")


@functools.partial(
    pl.kernel,
    mesh=_mesh(),
    out_type=jax.ShapeDtypeStruct((2, _NPAD), jnp.float32),
    scratch_types=[
        pltpu.VMEM((_ROWS, 128), jnp.int32),
        pltpu.VMEM((_ROWS, 128), jnp.int32),
        pltpu.VMEM((_ROWS, 128), jnp.float32),
        pltpu.VMEM_SHARED((_NPAD,), jnp.float32),
        pltpu.SemaphoreType.DMA,
        pltpu.SemaphoreType.DMA,
    ],
)
def _sc_deg(src_hbm, dst_hbm, zeros_hbm, degp_hbm,
            src_v, dst_v, val_v, acc_sh, dsem, ssem):
    c = lax.axis_index("c")
    s = lax.axis_index("s")
    w = c * 16 + s
    seg = pl.ds(s * (_NPAD // 16), _NPAD // 16)

    pltpu.async_copy(src_hbm.at[w], src_v, dsem)
    pltpu.async_copy(dst_hbm.at[w], dst_v, dsem)
    pltpu.sync_copy(zeros_hbm.at[seg], acc_sh.at[seg])
    plsc.subcore_barrier()
    pltpu.make_async_copy(src_hbm.at[w], src_v, dsem).wait()
    pltpu.make_async_copy(dst_hbm.at[w], dst_v, dsem).wait()

    def row(i, carry):
        src_r = src_v.at[i]
        dst_r = dst_v.at[i]
        val_r = val_v.at[i]

        def lane(j, carry2):
            sv = src_r[pl.ds(j * 16, 16)]
            dv = dst_r[pl.ds(j * 16, 16)]
            val_r[pl.ds(j * 16, 16)] = jnp.where(sv != dv, 1.0, 0.0)
            return carry2

        lax.fori_loop(0, 8, lane, carry, unroll=8)
        pltpu.async_copy(val_v.at[i], acc_sh.at[dst_v.at[i]], ssem, add=True)
        return carry

    lax.fori_loop(0, _ROWS, row, 0)

    def drain(i, carry):
        pltpu.make_async_copy(val_v.at[i], acc_sh.at[dst_v.at[i]], ssem).wait()
        return carry

    lax.fori_loop(0, _ROWS, drain, 0)
    plsc.subcore_barrier()
    pltpu.sync_copy(acc_sh.at[seg], degp_hbm.at[c].at[seg])


@functools.partial(
    pl.kernel,
    mesh=_mesh(),
    out_type=jax.ShapeDtypeStruct((2, _NPAD), jnp.float32),
    compiler_params=pltpu.CompilerParams(needs_layout_passes=False),
    scratch_types=[
        pltpu.VMEM((_ROWS, 128), jnp.int32),
        pltpu.VMEM((_ROWS, 128), jnp.int32),
        pltpu.VMEM((_ROWS, 128), jnp.float32),
        pltpu.VMEM((_NPAD,), jnp.float32),
        pltpu.VMEM_SHARED((_NPAD,), jnp.float32),
        pltpu.SemaphoreType.DMA,
        pltpu.SemaphoreType.DMA,
    ],
)
def _sc_agg(src_hbm, dst_hbm, z_hbm, zeros_hbm, outp_hbm,
            src_v, dst_v, val_v, z_v, acc_sh, dsem, ssem):
    c = lax.axis_index("c")
    s = lax.axis_index("s")
    w = c * 16 + s
    seg = pl.ds(s * (_NPAD // 16), _NPAD // 16)

    pltpu.async_copy(src_hbm.at[w], src_v, dsem)
    pltpu.async_copy(dst_hbm.at[w], dst_v, dsem)
    pltpu.async_copy(z_hbm, z_v, dsem)
    pltpu.sync_copy(zeros_hbm.at[seg], acc_sh.at[seg])
    plsc.subcore_barrier()
    pltpu.make_async_copy(src_hbm.at[w], src_v, dsem).wait()
    pltpu.make_async_copy(dst_hbm.at[w], dst_v, dsem).wait()
    pltpu.make_async_copy(z_hbm, z_v, dsem).wait()

    def row(i, carry):
        src_r = src_v.at[i]
        dst_r = dst_v.at[i]
        val_r = val_v.at[i]

        def lane(j, carry2):
            sv = src_r[pl.ds(j * 16, 16)]
            dv = dst_r[pl.ds(j * 16, 16)]
            zz = plsc.load_gather(z_v, [sv])
            val_r[pl.ds(j * 16, 16)] = jnp.where(sv != dv, zz, 0.0)
            return carry2

        lax.fori_loop(0, 8, lane, carry, unroll=8)
        pltpu.async_copy(val_v.at[i], acc_sh.at[dst_v.at[i]], ssem, add=True)
        return carry

    lax.fori_loop(0, _ROWS, row, 0)

    def drain(i, carry):
        pltpu.make_async_copy(val_v.at[i], acc_sh.at[dst_v.at[i]], ssem).wait()
        return carry

    lax.fori_loop(0, _ROWS, drain, 0)
    plsc.subcore_barrier()
    pltpu.sync_copy(acc_sh.at[seg], outp_hbm.at[c].at[seg])


def _tc_prep_body(degp_ref, xt_ref, w_ref, dinv_ref, z_ref):
    deg = degp_ref[0:1, :] + degp_ref[1:2, :]
    dinv = jnp.where(deg > 0.0, lax.rsqrt(deg), 0.0)
    y = jnp.sum(xt_ref[...] * w_ref[...], axis=0, keepdims=True)
    dinv_ref[...] = dinv
    z_ref[...] = dinv * y


_tc_prep = pl.pallas_call(
    _tc_prep_body,
    out_shape=(
        jax.ShapeDtypeStruct((1, _NPAD), jnp.float32),
        jax.ShapeDtypeStruct((1, _NPAD), jnp.float32),
    ),
)


def _tc_head_body(outp_ref, dinv_ref, temp_ref):
    o = (outp_ref[0:1, :] + outp_ref[1:2, :]) * dinv_ref[...]
    sp = jnp.maximum(o, 0.0) + jnp.log1p(jnp.exp(-jnp.abs(o))) + _TAU0
    temp_ref[...] = 1.0 / sp


_tc_head = pl.pallas_call(
    _tc_head_body,
    out_shape=jax.ShapeDtypeStruct((1, _NPAD), jnp.float32),
)


def kernel(x, edge_index, W):
    src = edge_index[0]
    dst = edge_index[1]
    pad = _EP - _E
    srcp = jnp.pad(src, (0, pad)).reshape(_NW, _ROWS, 128)
    dstp = jnp.pad(dst, (0, pad)).reshape(_NW, _ROWS, 128)
    zeros = jnp.zeros((_NPAD,), jnp.float32)

    degp = _sc_deg(srcp, dstp, zeros)
    xt = jnp.pad(x, ((0, _NPAD - _N), (0, 0))).T
    dinv, z = _tc_prep(degp, xt, W)
    outp = _sc_agg(srcp, dstp, z.reshape(_NPAD), zeros)
    temp = _tc_head(outp, dinv)
    return temp[0, :_N].reshape(_N, 1)


# speedup vs baseline: 102.9152x; 1.0198x over previous
import functools

import jax
import jax.numpy as jnp
from jax import lax
from jax.experimental import pallas as pl
from jax.experimental.pallas import tpu as pltpu
from jax.experimental.pallas import tpu_sc as plsc

_N = 10000
_E = 320000
_D = 128
_TAU0 = 0.1

_NPAD = 10240
_NW = 32
_ROWS = 80
_EPW = _ROWS * 128
_EP = _NW * _EPW


def _mesh():
    return plsc.VectorSubcoreMesh(core_axis_name="c", subcore_axis_name="s")


@functools.partial(
    pl.kernel,
    mesh=_mesh(),
    out_type=jax.ShapeDtypeStruct((2, _NPAD), jnp.float32),
    scratch_types=[
        pltpu.VMEM((_ROWS, 128), jnp.int32),
        pltpu.VMEM((_ROWS, 128), jnp.int32),
        pltpu.VMEM((_ROWS, 128), jnp.float32),
        pltpu.VMEM_SHARED((_NPAD,), jnp.float32),
        pltpu.SemaphoreType.DMA,
        pltpu.SemaphoreType.DMA,
    ],
)
def _sc_deg(src_hbm, dst_hbm, zeros_hbm, degp_hbm,
            src_v, dst_v, val_v, acc_sh, dsem, ssem):
    c = lax.axis_index("c")
    s = lax.axis_index("s")
    w = c * 16 + s
    seg = pl.ds(s * (_NPAD // 16), _NPAD // 16)

    pltpu.async_copy(src_hbm.at[w], src_v, dsem)
    pltpu.async_copy(dst_hbm.at[w], dst_v, dsem)
    pltpu.sync_copy(zeros_hbm.at[seg], acc_sh.at[seg])
    plsc.subcore_barrier()
    pltpu.make_async_copy(src_hbm.at[w], src_v, dsem).wait()
    pltpu.make_async_copy(dst_hbm.at[w], dst_v, dsem).wait()

    def row(i, carry):
        src_r = src_v.at[i]
        dst_r = dst_v.at[i]
        val_r = val_v.at[i]

        def lane(j, carry2):
            sv = src_r[pl.ds(j * 16, 16)]
            dv = dst_r[pl.ds(j * 16, 16)]
            val_r[pl.ds(j * 16, 16)] = jnp.where(sv != dv, 1.0, 0.0)
            return carry2

        lax.fori_loop(0, 8, lane, carry, unroll=8)
        pltpu.async_copy(val_v.at[i], acc_sh.at[dst_v.at[i]], ssem, add=True)
        return carry

    lax.fori_loop(0, _ROWS, row, 0, unroll=4)

    def drain(i, carry):
        pltpu.make_async_copy(val_v.at[i], acc_sh.at[dst_v.at[i]], ssem).wait()
        return carry

    lax.fori_loop(0, _ROWS, drain, 0)
    plsc.subcore_barrier()
    pltpu.sync_copy(acc_sh.at[seg], degp_hbm.at[c].at[seg])


@functools.partial(
    pl.kernel,
    mesh=_mesh(),
    out_type=jax.ShapeDtypeStruct((2, _NPAD), jnp.float32),
    compiler_params=pltpu.CompilerParams(needs_layout_passes=False),
    scratch_types=[
        pltpu.VMEM((_ROWS, 128), jnp.int32),
        pltpu.VMEM((_ROWS, 128), jnp.int32),
        pltpu.VMEM((_ROWS, 128), jnp.float32),
        pltpu.VMEM((_NPAD,), jnp.float32),
        pltpu.VMEM_SHARED((_NPAD,), jnp.float32),
        pltpu.SemaphoreType.DMA,
        pltpu.SemaphoreType.DMA,
    ],
)
def _sc_agg(src_hbm, dst_hbm, z_hbm, zeros_hbm, outp_hbm,
            src_v, dst_v, val_v, z_v, acc_sh, dsem, ssem):
    c = lax.axis_index("c")
    s = lax.axis_index("s")
    w = c * 16 + s
    seg = pl.ds(s * (_NPAD // 16), _NPAD // 16)

    pltpu.async_copy(src_hbm.at[w], src_v, dsem)
    pltpu.async_copy(dst_hbm.at[w], dst_v, dsem)
    pltpu.async_copy(z_hbm, z_v, dsem)
    pltpu.sync_copy(zeros_hbm.at[seg], acc_sh.at[seg])
    plsc.subcore_barrier()
    pltpu.make_async_copy(src_hbm.at[w], src_v, dsem).wait()
    pltpu.make_async_copy(dst_hbm.at[w], dst_v, dsem).wait()
    pltpu.make_async_copy(z_hbm, z_v, dsem).wait()

    def row(i, carry):
        src_r = src_v.at[i]
        dst_r = dst_v.at[i]
        val_r = val_v.at[i]

        def lane(j, carry2):
            sv = src_r[pl.ds(j * 16, 16)]
            dv = dst_r[pl.ds(j * 16, 16)]
            zz = plsc.load_gather(z_v, [sv])
            val_r[pl.ds(j * 16, 16)] = jnp.where(sv != dv, zz, 0.0)
            return carry2

        lax.fori_loop(0, 8, lane, carry, unroll=8)
        pltpu.async_copy(val_v.at[i], acc_sh.at[dst_v.at[i]], ssem, add=True)
        return carry

    lax.fori_loop(0, _ROWS, row, 0, unroll=4)

    def drain(i, carry):
        pltpu.make_async_copy(val_v.at[i], acc_sh.at[dst_v.at[i]], ssem).wait()
        return carry

    lax.fori_loop(0, _ROWS, drain, 0)
    plsc.subcore_barrier()
    pltpu.sync_copy(acc_sh.at[seg], outp_hbm.at[c].at[seg])


def _tc_matvec_body(xt_ref, w_ref, y_ref):
    y_ref[...] = jnp.sum(xt_ref[...] * w_ref[...], axis=0, keepdims=True)


_tc_matvec = pl.pallas_call(
    _tc_matvec_body,
    out_shape=jax.ShapeDtypeStruct((1, _NPAD), jnp.float32),
)


def _tc_prep_body(degp_ref, y_ref, dinv_ref, z_ref):
    deg = degp_ref[0:1, :] + degp_ref[1:2, :]
    dinv = jnp.where(deg > 0.0, lax.rsqrt(deg), 0.0)
    dinv_ref[...] = dinv
    z_ref[...] = dinv * y_ref[...]


_tc_prep = pl.pallas_call(
    _tc_prep_body,
    out_shape=(
        jax.ShapeDtypeStruct((1, _NPAD), jnp.float32),
        jax.ShapeDtypeStruct((1, _NPAD), jnp.float32),
    ),
)


def _tc_head_body(outp_ref, dinv_ref, temp_ref):
    o = (outp_ref[0:1, :] + outp_ref[1:2, :]) * dinv_ref[...]
    sp = jnp.maximum(o, 0.0) + jnp.log1p(jnp.exp(-jnp.abs(o))) + _TAU0
    temp_ref[...] = 1.0 / sp


_tc_head = pl.pallas_call(
    _tc_head_body,
    out_shape=jax.ShapeDtypeStruct((1, _NPAD), jnp.float32),
)


def kernel(x, edge_index, W):
    src = edge_index[0]
    dst = edge_index[1]
    pad = _EP - _E
    srcp = jnp.pad(src, (0, pad)).reshape(_NW, _ROWS, 128)
    dstp = jnp.pad(dst, (0, pad)).reshape(_NW, _ROWS, 128)
    zeros = jnp.zeros((_NPAD,), jnp.float32)

    xt = jnp.pad(x, ((0, _NPAD - _N), (0, 0))).T
    y = _tc_matvec(xt, W)
    degp = _sc_deg(srcp, dstp, zeros)
    dinv, z = _tc_prep(degp, y)
    outp = _sc_agg(srcp, dstp, z.reshape(_NPAD), zeros)
    temp = _tc_head(outp, dinv)
    return temp[0, :_N].reshape(_N, 1)


# speedup vs baseline: 104.8021x; 1.0183x over previous
import functools

import jax
import jax.numpy as jnp
from jax import lax
from jax.experimental import pallas as pl
from jax.experimental.pallas import tpu as pltpu
from jax.experimental.pallas import tpu_sc as plsc

_N = 10000
_E = 320000
_D = 128
_TAU0 = 0.1

_NPAD = 10240
_NW = 32
_CH = 10240
_EP = _NW * _CH
_ITERS = _CH // 16


def _mesh():
    return plsc.VectorSubcoreMesh(core_axis_name="c", subcore_axis_name="s")


@functools.partial(
    pl.kernel,
    mesh=_mesh(),
    out_type=jax.ShapeDtypeStruct((2, _NPAD), jnp.float32),
    scratch_types=[
        pltpu.VMEM((_CH,), jnp.int32),
        pltpu.VMEM((_CH,), jnp.int32),
        pltpu.VMEM((_CH,), jnp.float32),
        pltpu.VMEM_SHARED((_NPAD,), jnp.float32),
        pltpu.SemaphoreType.DMA,
        pltpu.SemaphoreType.DMA,
    ],
)
def _sc_deg(ei_hbm, zeros_hbm, degp_hbm,
            src_v, dst_v, val_v, acc_sh, dsem, ssem):
    c = lax.axis_index("c")
    s = lax.axis_index("s")
    w = c * 16 + s
    base = w * _CH
    seg = pl.ds(s * (_NPAD // 16), _NPAD // 16)

    pltpu.async_copy(ei_hbm.at[0].at[pl.ds(base, _CH)], src_v, dsem)
    pltpu.async_copy(ei_hbm.at[1].at[pl.ds(base, _CH)], dst_v, dsem)
    pltpu.sync_copy(zeros_hbm.at[seg], acc_sh.at[seg])
    plsc.subcore_barrier()
    pltpu.make_async_copy(ei_hbm.at[0].at[pl.ds(base, _CH)], src_v, dsem).wait()
    pltpu.make_async_copy(ei_hbm.at[1].at[pl.ds(base, _CH)], dst_v, dsem).wait()

    def lane(k, carry):
        sl = pl.ds(k * 16, 16)
        sv = src_v[sl]
        dv = dst_v[sl]
        val_v[sl] = jnp.where(sv != dv, 1.0, 0.0)
        return carry

    lax.fori_loop(0, _ITERS, lane, 0, unroll=8)

    pltpu.sync_copy(val_v, acc_sh.at[dst_v], add=True)
    plsc.subcore_barrier()
    pltpu.sync_copy(acc_sh.at[seg], degp_hbm.at[c].at[seg])


@functools.partial(
    pl.kernel,
    mesh=_mesh(),
    out_type=jax.ShapeDtypeStruct((2, _NPAD), jnp.float32),
    compiler_params=pltpu.CompilerParams(needs_layout_passes=False),
    scratch_types=[
        pltpu.VMEM((_CH,), jnp.int32),
        pltpu.VMEM((_CH,), jnp.int32),
        pltpu.VMEM((_CH,), jnp.float32),
        pltpu.VMEM((_NPAD,), jnp.float32),
        pltpu.VMEM_SHARED((_NPAD,), jnp.float32),
        pltpu.SemaphoreType.DMA,
        pltpu.SemaphoreType.DMA,
    ],
)
def _sc_agg(ei_hbm, z_hbm, zeros_hbm, outp_hbm,
            src_v, dst_v, val_v, z_v, acc_sh, dsem, ssem):
    c = lax.axis_index("c")
    s = lax.axis_index("s")
    w = c * 16 + s
    base = w * _CH
    seg = pl.ds(s * (_NPAD // 16), _NPAD // 16)

    pltpu.async_copy(ei_hbm.at[0].at[pl.ds(base, _CH)], src_v, dsem)
    pltpu.async_copy(ei_hbm.at[1].at[pl.ds(base, _CH)], dst_v, dsem)
    pltpu.async_copy(z_hbm, z_v, dsem)
    pltpu.sync_copy(zeros_hbm.at[seg], acc_sh.at[seg])
    plsc.subcore_barrier()
    pltpu.make_async_copy(ei_hbm.at[0].at[pl.ds(base, _CH)], src_v, dsem).wait()
    pltpu.make_async_copy(ei_hbm.at[1].at[pl.ds(base, _CH)], dst_v, dsem).wait()
    pltpu.make_async_copy(z_hbm, z_v, dsem).wait()

    def lane(k, carry):
        sl = pl.ds(k * 16, 16)
        sv = src_v[sl]
        dv = dst_v[sl]
        zz = plsc.load_gather(z_v, [sv])
        val_v[sl] = jnp.where(sv != dv, zz, 0.0)
        return carry

    lax.fori_loop(0, _ITERS, lane, 0, unroll=8)

    pltpu.sync_copy(val_v, acc_sh.at[dst_v], add=True)
    plsc.subcore_barrier()
    pltpu.sync_copy(acc_sh.at[seg], outp_hbm.at[c].at[seg])


def _tc_matvec_body(xt_ref, w_ref, y_ref):
    y_ref[...] = jnp.sum(xt_ref[...] * w_ref[...], axis=0, keepdims=True)


_tc_matvec = pl.pallas_call(
    _tc_matvec_body,
    out_shape=jax.ShapeDtypeStruct((1, _NPAD), jnp.float32),
)


def _tc_prep_body(degp_ref, y_ref, dinv_ref, z_ref):
    deg = degp_ref[0:1, :] + degp_ref[1:2, :]
    dinv = jnp.where(deg > 0.0, lax.rsqrt(deg), 0.0)
    dinv_ref[...] = dinv
    z_ref[...] = dinv * y_ref[...]


_tc_prep = pl.pallas_call(
    _tc_prep_body,
    out_shape=(
        jax.ShapeDtypeStruct((1, _NPAD), jnp.float32),
        jax.ShapeDtypeStruct((1, _NPAD), jnp.float32),
    ),
)


def _tc_head_body(outp_ref, dinv_ref, temp_ref):
    o = (outp_ref[0:1, :] + outp_ref[1:2, :]) * dinv_ref[...]
    sp = jnp.maximum(o, 0.0) + jnp.log1p(jnp.exp(-jnp.abs(o))) + _TAU0
    temp_ref[...] = 1.0 / sp


_tc_head = pl.pallas_call(
    _tc_head_body,
    out_shape=jax.ShapeDtypeStruct((1, _NPAD), jnp.float32),
)


def kernel(x, edge_index, W):
    zeros = jnp.zeros((_NPAD,), jnp.float32)
    ei = jnp.pad(edge_index, ((0, 0), (0, _EP - _E)))
    xt = jnp.pad(x, ((0, _NPAD - _N), (0, 0))).T
    y = _tc_matvec(xt, W)
    degp = _sc_deg(ei, zeros)
    dinv, z = _tc_prep(degp, y)
    outp = _sc_agg(ei, z.reshape(_NPAD), zeros)
    temp = _tc_head(outp, dinv)
    return temp[0, :_N].reshape(_N, 1)


# speedup vs baseline: 116.1105x; 1.1079x over previous
import functools

import jax
import jax.numpy as jnp
from jax import lax
from jax.experimental import pallas as pl
from jax.experimental.pallas import tpu as pltpu
from jax.experimental.pallas import tpu_sc as plsc

_N = 10000
_E = 320000
_D = 128
_TAU0 = 0.1

_NPAD = 10240
_NW = 32
_EPW = _E // _NW
_WIN = 10240
_ITERS = _EPW // 16
_SEG = _NPAD // 16


def _mesh():
    return plsc.VectorSubcoreMesh(core_axis_name="c", subcore_axis_name="s")


@functools.partial(
    pl.kernel,
    mesh=_mesh(),
    out_type=jax.ShapeDtypeStruct((2, _NPAD), jnp.float32),
    scratch_types=[
        pltpu.VMEM((_WIN,), jnp.int32),
        pltpu.VMEM((_WIN,), jnp.int32),
        pltpu.VMEM((_WIN,), jnp.float32),
        pltpu.VMEM((16 * _SEG,), jnp.float32),
        pltpu.VMEM((_SEG,), jnp.float32),
        pltpu.VMEM_SHARED((16 * _NPAD,), jnp.float32),
        pltpu.SemaphoreType.DMA,
        pltpu.SemaphoreType.DMA,
    ],
)
def _sc_deg(ei_hbm, zeros_hbm, degp_hbm,
            src_v, dst_v, val_v, comb_v, res_v, parts_sh, dsem, ssem):
    c = lax.axis_index("c")
    s = lax.axis_index("s")
    w = c * 16 + s
    base = w * _EPW
    base_al = jnp.minimum((base // 128) * 128, _E - _WIN)
    base_al = pl.multiple_of(base_al, 128)
    off = base - base_al
    nh = off // 16
    seg = pl.ds(s * (_NPAD // 16), _NPAD // 16)

    pltpu.async_copy(ei_hbm.at[0].at[pl.ds(base_al, _WIN)], src_v, dsem)
    pltpu.async_copy(ei_hbm.at[1].at[pl.ds(base_al, _WIN)], dst_v, dsem)
    pltpu.sync_copy(zeros_hbm, parts_sh.at[pl.ds(s * _NPAD, _NPAD)])
    pltpu.make_async_copy(ei_hbm.at[0].at[pl.ds(base_al, _WIN)], src_v, dsem).wait()
    pltpu.make_async_copy(ei_hbm.at[1].at[pl.ds(base_al, _WIN)], dst_v, dsem).wait()

    my0 = s * _NPAD
    zi = jnp.full((16,), 0, jnp.int32)
    zf = jnp.zeros((16,), jnp.float32)

    def headz(j, carry):
        sl = pl.ds(j * 16, 16)
        dst_v[sl] = zi + my0
        val_v[sl] = zf
        return carry

    lax.fori_loop(0, nh, headz, 0)

    def tailz(j, carry):
        sl = pl.ds(off + _EPW + j * 16, 16)
        dst_v[sl] = zi + my0
        val_v[sl] = zf
        return carry

    lax.fori_loop(0, 15 - nh, tailz, 0)

    def lane(k, carry):
        sl = pl.ds(off + k * 16, 16)
        sv = src_v[sl]
        dv = dst_v[sl]
        val_v[sl] = jnp.where(sv != dv, 1.0, 0.0)
        dst_v[sl] = dv + my0
        return carry

    lax.fori_loop(0, _ITERS, lane, 0, unroll=8)

    pltpu.sync_copy(val_v, parts_sh.at[dst_v], add=True)
    plsc.subcore_barrier()

    for t in range(16):
        pltpu.async_copy(
            parts_sh.at[pl.ds(t * _NPAD + s * _SEG, _SEG)],
            comb_v.at[pl.ds(t * _SEG, _SEG)], dsem)
    for t in range(16):
        pltpu.make_async_copy(
            parts_sh.at[pl.ds(t * _NPAD + s * _SEG, _SEG)],
            comb_v.at[pl.ds(t * _SEG, _SEG)], dsem).wait()

    def red(q, carry):
        acc = comb_v[pl.ds(q * 16, 16)]
        for t in range(1, 16):
            acc = acc + comb_v[pl.ds(t * _SEG + q * 16, 16)]
        res_v[pl.ds(q * 16, 16)] = acc
        return carry

    lax.fori_loop(0, _SEG // 16, red, 0, unroll=4)
    pltpu.sync_copy(res_v, degp_hbm.at[c].at[seg])


@functools.partial(
    pl.kernel,
    mesh=_mesh(),
    out_type=jax.ShapeDtypeStruct((2, _NPAD), jnp.float32),
    compiler_params=pltpu.CompilerParams(needs_layout_passes=False),
    scratch_types=[
        pltpu.VMEM((_WIN,), jnp.int32),
        pltpu.VMEM((_WIN,), jnp.int32),
        pltpu.VMEM((_WIN,), jnp.float32),
        pltpu.VMEM((_NPAD,), jnp.float32),
        pltpu.VMEM((16 * _SEG,), jnp.float32),
        pltpu.VMEM((_SEG,), jnp.float32),
        pltpu.VMEM_SHARED((16 * _NPAD,), jnp.float32),
        pltpu.SemaphoreType.DMA,
        pltpu.SemaphoreType.DMA,
    ],
)
def _sc_agg(ei_hbm, z_hbm, zeros_hbm, outp_hbm,
            src_v, dst_v, val_v, z_v, comb_v, res_v, parts_sh, dsem, ssem):
    c = lax.axis_index("c")
    s = lax.axis_index("s")
    w = c * 16 + s
    base = w * _EPW
    base_al = jnp.minimum((base // 128) * 128, _E - _WIN)
    base_al = pl.multiple_of(base_al, 128)
    off = base - base_al
    nh = off // 16
    seg = pl.ds(s * (_NPAD // 16), _NPAD // 16)

    pltpu.async_copy(ei_hbm.at[0].at[pl.ds(base_al, _WIN)], src_v, dsem)
    pltpu.async_copy(ei_hbm.at[1].at[pl.ds(base_al, _WIN)], dst_v, dsem)
    pltpu.async_copy(z_hbm, z_v, dsem)
    pltpu.sync_copy(zeros_hbm, parts_sh.at[pl.ds(s * _NPAD, _NPAD)])
    pltpu.make_async_copy(ei_hbm.at[0].at[pl.ds(base_al, _WIN)], src_v, dsem).wait()
    pltpu.make_async_copy(ei_hbm.at[1].at[pl.ds(base_al, _WIN)], dst_v, dsem).wait()
    pltpu.make_async_copy(z_hbm, z_v, dsem).wait()

    my0 = s * _NPAD
    zi = jnp.full((16,), 0, jnp.int32)
    zf = jnp.zeros((16,), jnp.float32)

    def headz(j, carry):
        sl = pl.ds(j * 16, 16)
        dst_v[sl] = zi + my0
        val_v[sl] = zf
        return carry

    lax.fori_loop(0, nh, headz, 0)

    def tailz(j, carry):
        sl = pl.ds(off + _EPW + j * 16, 16)
        dst_v[sl] = zi + my0
        val_v[sl] = zf
        return carry

    lax.fori_loop(0, 15 - nh, tailz, 0)

    def lane(k, carry):
        sl = pl.ds(off + k * 16, 16)
        sv = src_v[sl]
        dv = dst_v[sl]
        zz = plsc.load_gather(z_v, [sv])
        val_v[sl] = jnp.where(sv != dv, zz, 0.0)
        dst_v[sl] = dv + my0
        return carry

    lax.fori_loop(0, _ITERS, lane, 0, unroll=8)

    pltpu.sync_copy(val_v, parts_sh.at[dst_v], add=True)
    plsc.subcore_barrier()

    for t in range(16):
        pltpu.async_copy(
            parts_sh.at[pl.ds(t * _NPAD + s * _SEG, _SEG)],
            comb_v.at[pl.ds(t * _SEG, _SEG)], dsem)
    for t in range(16):
        pltpu.make_async_copy(
            parts_sh.at[pl.ds(t * _NPAD + s * _SEG, _SEG)],
            comb_v.at[pl.ds(t * _SEG, _SEG)], dsem).wait()

    def red(q, carry):
        acc = comb_v[pl.ds(q * 16, 16)]
        for t in range(1, 16):
            acc = acc + comb_v[pl.ds(t * _SEG + q * 16, 16)]
        res_v[pl.ds(q * 16, 16)] = acc
        return carry

    lax.fori_loop(0, _SEG // 16, red, 0, unroll=4)
    pltpu.sync_copy(res_v, outp_hbm.at[c].at[seg])


def _tc_matvec_body(xt_ref, w_ref, y_ref):
    y_ref[...] = jnp.sum(xt_ref[...] * w_ref[...], axis=0, keepdims=True)


_tc_matvec = pl.pallas_call(
    _tc_matvec_body,
    out_shape=jax.ShapeDtypeStruct((1, _NPAD), jnp.float32),
)


def _tc_prep_body(degp_ref, y_ref, dinv_ref, z_ref):
    deg = degp_ref[0:1, :] + degp_ref[1:2, :]
    dinv = jnp.where(deg > 0.0, lax.rsqrt(deg), 0.0)
    dinv_ref[...] = dinv
    z_ref[...] = dinv * y_ref[...]


_tc_prep = pl.pallas_call(
    _tc_prep_body,
    out_shape=(
        jax.ShapeDtypeStruct((1, _NPAD), jnp.float32),
        jax.ShapeDtypeStruct((1, _NPAD), jnp.float32),
    ),
)


def _tc_head_body(outp_ref, dinv_ref, temp_ref):
    o = (outp_ref[0:1, :] + outp_ref[1:2, :]) * dinv_ref[...]
    sp = jnp.maximum(o, 0.0) + jnp.log1p(jnp.exp(-jnp.abs(o))) + _TAU0
    temp_ref[...] = 1.0 / sp


_tc_head = pl.pallas_call(
    _tc_head_body,
    out_shape=jax.ShapeDtypeStruct((1, _NPAD), jnp.float32),
)


def kernel(x, edge_index, W):
    zeros = jnp.zeros((_NPAD,), jnp.float32)
    xt = jnp.pad(x, ((0, _NPAD - _N), (0, 0))).T
    y = _tc_matvec(xt, W)
    degp = _sc_deg(edge_index, zeros)
    dinv, z = _tc_prep(degp, y)
    outp = _sc_agg(edge_index, z.reshape(_NPAD), zeros)
    temp = _tc_head(outp, dinv)
    return temp[0, :_N].reshape(_N, 1)
